# baseline (speedup 1.0000x reference)
import jax
import jax.numpy as jnp
from jax.experimental import pallas as pl
from jax.experimental.pallas import tpu as pltpu


def _cell_kernel(x_ref, h_ref, wz_ref, wr_ref, wh_ref, bz_ref, br_ref,
                 bh_ref, lin_ref, linb_ref, out_ref, H_ref):
    x = x_ref[...]
    h = h_ref[...]
    hb = h
    F = x.shape[1]

    wz = wz_ref[...]
    wr = wr_ref[...]
    wh = wh_ref[...]
    cin = wz.shape[0] // 2
    Wz = wz[:cin] + wz[cin:]
    Wr = wr[:cin] + wr[cin:]
    Wh = wh[:cin] + wh[cin:]

    f32 = jnp.float32
    z = jax.nn.sigmoid(jnp.dot(x, Wz[:F], preferred_element_type=f32) +
                       jnp.dot(hb, Wz[F:], preferred_element_type=f32) +
                       bz_ref[...])
    r = jax.nn.sigmoid(jnp.dot(x, Wr[:F], preferred_element_type=f32) +
                       jnp.dot(hb, Wr[F:], preferred_element_type=f32) +
                       br_ref[...])
    rh = r * h
    ht = jnp.tanh(jnp.dot(x, Wh[:F], preferred_element_type=f32) +
                  jnp.dot(rh, Wh[F:], preferred_element_type=f32) +
                  bh_ref[...])
    H = z * h + (1.0 - z) * ht
    H_ref[...] = H
    out_ref[...] = (jnp.sum(jnp.maximum(H, 0.0) * lin_ref[...],
                            axis=1, keepdims=True) + linb_ref[...])


def kernel(x, edge_index, edge_attr, h, W_z, b_z, W_r, b_r, W_h, b_h,
           lin_w, lin_b):
    del edge_index, edge_attr
    N, F = x.shape
    D = h.shape[1]
    cin = F + D

    wz = W_z.reshape(2 * cin, D)
    wr = W_r.reshape(2 * cin, D)
    wh = W_h.reshape(2 * cin, D)
    bz = b_z.reshape(1, D)
    br = b_r.reshape(1, D)
    bh = b_h.reshape(1, D)
    lin = lin_w.reshape(1, D)
    linb = lin_b.reshape(1, 1)

    B = 10000
    grid = (N // B,)

    out, H = pl.pallas_call(
        _cell_kernel,
        grid=grid,
        in_specs=[
            pl.BlockSpec((B, F), lambda i: (i, 0)),
            pl.BlockSpec((B, D), lambda i: (i, 0)),
            pl.BlockSpec((2 * cin, D), lambda i: (0, 0)),
            pl.BlockSpec((2 * cin, D), lambda i: (0, 0)),
            pl.BlockSpec((2 * cin, D), lambda i: (0, 0)),
            pl.BlockSpec((1, D), lambda i: (0, 0)),
            pl.BlockSpec((1, D), lambda i: (0, 0)),
            pl.BlockSpec((1, D), lambda i: (0, 0)),
            pl.BlockSpec((1, D), lambda i: (0, 0)),
            pl.BlockSpec((1, 1), lambda i: (0, 0)),
        ],
        out_specs=[
            pl.BlockSpec((B, 1), lambda i: (i, 0)),
            pl.BlockSpec((B, D), lambda i: (i, 0)),
        ],
        out_shape=[
            jax.ShapeDtypeStruct((N, 1), jnp.float32),
            jax.ShapeDtypeStruct((N, D), jnp.float32),
        ],
        compiler_params=pltpu.CompilerParams(
            dimension_semantics=("parallel",),
        ),
    )(x, h, wz, wr, wh, bz, br, bh, lin, linb)
    return (out, H)


# speedup vs baseline: 1.1308x; 1.1308x over previous
import jax
import jax.numpy as jnp
from jax.experimental import pallas as pl
from jax.experimental.pallas import tpu as pltpu

_NBUF = 4
_BC = 1000


def _cell_body(x_hbm, h_hbm, wz_ref, wr_ref, wh_ref, bz_ref, br_ref, bh_ref,
               lin_ref, linb_ref, out_hbm, H_hbm,
               x_buf, h_buf, H_buf, o_buf, in_sems, out_sems):
    N, F = x_hbm.shape
    nc = N // _BC

    wz = wz_ref[...]
    wr = wr_ref[...]
    wh = wh_ref[...]
    cin = wz.shape[0] // 2
    Wz = wz[:cin] + wz[cin:]
    Wr = wr[:cin] + wr[cin:]
    Wh = wh[:cin] + wh[cin:]
    bz = bz_ref[...]
    br = br_ref[...]
    bh = bh_ref[...]
    lin = lin_ref[...]
    linb = linb_ref[...]

    def in_copies(i):
        slot = i % _NBUF
        rows = pl.ds(i * _BC, _BC)
        return (pltpu.make_async_copy(x_hbm.at[rows, :], x_buf.at[slot],
                                      in_sems.at[slot, 0]),
                pltpu.make_async_copy(h_hbm.at[rows, :], h_buf.at[slot],
                                      in_sems.at[slot, 1]))

    def out_copies(i):
        slot = i % _NBUF
        rows = pl.ds(i * _BC, _BC)
        return (pltpu.make_async_copy(H_buf.at[slot], H_hbm.at[rows, :],
                                      out_sems.at[slot, 0]),
                pltpu.make_async_copy(o_buf.at[slot], out_hbm.at[rows, :],
                                      out_sems.at[slot, 1]))

    for i in range(min(_NBUF, nc)):
        for c in in_copies(i):
            c.start()

    f32 = jnp.float32
    for i in range(nc):
        slot = i % _NBUF
        for c in in_copies(i):
            c.wait()
        if i >= _NBUF:
            for c in out_copies(i - _NBUF):
                c.wait()
        x = x_buf[slot]
        h = h_buf[slot]
        z = jax.nn.sigmoid(jnp.dot(x, Wz[:F], preferred_element_type=f32) +
                           jnp.dot(h, Wz[F:], preferred_element_type=f32) +
                           bz)
        r = jax.nn.sigmoid(jnp.dot(x, Wr[:F], preferred_element_type=f32) +
                           jnp.dot(h, Wr[F:], preferred_element_type=f32) +
                           br)
        ht = jnp.tanh(jnp.dot(x, Wh[:F], preferred_element_type=f32) +
                      jnp.dot(r * h, Wh[F:], preferred_element_type=f32) +
                      bh)
        H = z * h + (1.0 - z) * ht
        H_buf[slot] = H
        o_buf[slot] = (jnp.sum(jnp.maximum(H, 0.0) * lin, axis=1,
                               keepdims=True) + linb)
        for c in out_copies(i):
            c.start()
        if i + _NBUF < nc:
            for c in in_copies(i + _NBUF):
                c.start()

    for i in range(max(0, nc - _NBUF), nc):
        for c in out_copies(i):
            c.wait()


def kernel(x, edge_index, edge_attr, h, W_z, b_z, W_r, b_r, W_h, b_h,
           lin_w, lin_b):
    del edge_index, edge_attr
    N, F = x.shape
    D = h.shape[1]
    cin = F + D

    wz = W_z.reshape(2 * cin, D)
    wr = W_r.reshape(2 * cin, D)
    wh = W_h.reshape(2 * cin, D)
    bz = b_z.reshape(1, D)
    br = b_r.reshape(1, D)
    bh = b_h.reshape(1, D)
    lin = lin_w.reshape(1, D)
    linb = lin_b.reshape(1, 1)

    hbm = pltpu.MemorySpace.HBM
    vmem = pltpu.MemorySpace.VMEM

    out, H = pl.pallas_call(
        _cell_body,
        in_specs=[
            pl.BlockSpec(memory_space=hbm),
            pl.BlockSpec(memory_space=hbm),
            pl.BlockSpec(memory_space=vmem),
            pl.BlockSpec(memory_space=vmem),
            pl.BlockSpec(memory_space=vmem),
            pl.BlockSpec(memory_space=vmem),
            pl.BlockSpec(memory_space=vmem),
            pl.BlockSpec(memory_space=vmem),
            pl.BlockSpec(memory_space=vmem),
            pl.BlockSpec(memory_space=vmem),
        ],
        out_specs=[
            pl.BlockSpec(memory_space=hbm),
            pl.BlockSpec(memory_space=hbm),
        ],
        out_shape=[
            jax.ShapeDtypeStruct((N, 1), jnp.float32),
            jax.ShapeDtypeStruct((N, D), jnp.float32),
        ],
        scratch_shapes=[
            pltpu.VMEM((_NBUF, _BC, F), jnp.float32),
            pltpu.VMEM((_NBUF, _BC, D), jnp.float32),
            pltpu.VMEM((_NBUF, _BC, D), jnp.float32),
            pltpu.VMEM((_NBUF, _BC, 1), jnp.float32),
            pltpu.SemaphoreType.DMA((_NBUF, 2)),
            pltpu.SemaphoreType.DMA((_NBUF, 2)),
        ],
    )(x, h, wz, wr, wh, bz, br, bh, lin, linb)
    return (out, H)
